# baseline (speedup 1.0000x reference)
import functools

import jax
import jax.numpy as jnp
from jax import lax
from jax.experimental import pallas as pl
from jax.experimental.pallas import tpu as pltpu
from jax.experimental.pallas import tpu_sc as plsc

NC = 2
NS = 16
NW = NC * NS
CHUNK = 128
PACK = 4


@functools.lru_cache(maxsize=None)
def _make_gather(batch, nu_sup, nm_sup):
    bpw = batch // NW
    nchunks = bpw // CHUNK
    mesh = plsc.VectorSubcoreMesh(core_axis_name="c", subcore_axis_name="s")

    @functools.partial(
        pl.kernel,
        mesh=mesh,
        out_type=[
            jax.ShapeDtypeStruct((batch, 128), jnp.float32),
            jax.ShapeDtypeStruct((batch, 128), jnp.float32),
        ],
        scratch_types=[
            pltpu.VMEM((nchunks, CHUNK), jnp.int32),
            pltpu.VMEM((nchunks, CHUNK), jnp.int32),
            pltpu.VMEM((bpw, 128), jnp.float32),
            pltpu.SemaphoreType.DMA,
        ],
    )
    def gather_k(uid_hbm, mid_hbm, utab_hbm, mtab_hbm, uout_hbm, mout_hbm,
                 uidx_v, midx_v, rows_v, sem):
        wid = lax.axis_index("s") * NC + lax.axis_index("c")
        base = wid * bpw
        pltpu.sync_copy(uid_hbm.at[wid], uidx_v)
        pltpu.sync_copy(mid_hbm.at[wid], midx_v)
        copies = []
        for c in range(nchunks):
            copies.append(pltpu.async_copy(
                utab_hbm.at[uidx_v.at[c]],
                rows_v.at[pl.ds(c * CHUNK, CHUNK)], sem))
        for cp in copies:
            cp.wait()
        pltpu.sync_copy(rows_v, uout_hbm.at[pl.ds(base, bpw)])
        copies = []
        for c in range(nchunks):
            copies.append(pltpu.async_copy(
                mtab_hbm.at[midx_v.at[c]],
                rows_v.at[pl.ds(c * CHUNK, CHUNK)], sem))
        for cp in copies:
            cp.wait()
        pltpu.sync_copy(rows_v, mout_hbm.at[pl.ds(base, bpw)])

    return gather_k


def _head_body(u_ref, m_ref, urem_ref, mrem_ref, g_ref, gw_ref, gb_ref,
               fcw_ref, fcb_ref, o_ref):
    d = gw_ref.shape[0]
    seg = lax.broadcasted_iota(jnp.int32, (1, PACK * d), 1) // d
    umask = (seg == urem_ref[...]).astype(jnp.float32)
    mmask = (seg == mrem_ref[...]).astype(jnp.float32)
    fcw = fcw_ref[...]
    wu = fcw[:, 0:d]
    wm = fcw[:, d:2 * d]
    wg = fcw[:, 2 * d:3 * d]
    wu_t = jnp.concatenate([wu] * PACK, axis=1).T
    wm_t = jnp.concatenate([wm] * PACK, axis=1).T
    genre_emb = jnp.dot(g_ref[...], gw_ref[...].T,
                        preferred_element_type=jnp.float32) + gb_ref[...]
    o_ref[...] = (
        jnp.dot(u_ref[...] * umask, wu_t, preferred_element_type=jnp.float32)
        + jnp.dot(m_ref[...] * mmask, wm_t, preferred_element_type=jnp.float32)
        + jnp.dot(genre_emb, wg.T, preferred_element_type=jnp.float32)
        + fcb_ref[...]
    )


def kernel(user_id, movie_id, genre_features, user_table, movie_table,
           genre_W, genre_b, fc_W, fc_b):
    batch = user_id.shape[0]
    bpw = batch // NW
    nchunks = bpw // CHUNK

    uid = user_id.astype(jnp.int32)
    mid = movie_id.astype(jnp.int32)
    usup = (uid // PACK).reshape(NW, nchunks, CHUNK)
    msup = (mid // PACK).reshape(NW, nchunks, CHUNK)
    urem = (uid % PACK).reshape(batch, 1)
    mrem = (mid % PACK).reshape(batch, 1)

    utab = user_table.reshape(-1, 128)
    mtab = movie_table.reshape(-1, 128)

    u_full, m_full = _make_gather(batch, utab.shape[0], mtab.shape[0])(
        usup, msup, utab, mtab)

    blk = 4096
    gd = genre_features.shape[1]
    ed = user_table.shape[1]
    head = pl.pallas_call(
        _head_body,
        grid=(batch // blk,),
        in_specs=[
            pl.BlockSpec((blk, PACK * ed), lambda i: (i, 0)),
            pl.BlockSpec((blk, PACK * ed), lambda i: (i, 0)),
            pl.BlockSpec((blk, 1), lambda i: (i, 0)),
            pl.BlockSpec((blk, 1), lambda i: (i, 0)),
            pl.BlockSpec((blk, gd), lambda i: (i, 0)),
            pl.BlockSpec((ed, gd), lambda i: (0, 0)),
            pl.BlockSpec((1, ed), lambda i: (0, 0)),
            pl.BlockSpec(fc_W.shape, lambda i: (0, 0)),
            pl.BlockSpec((1, 1), lambda i: (0, 0)),
        ],
        out_specs=pl.BlockSpec((blk, 1), lambda i: (i, 0)),
        out_shape=jax.ShapeDtypeStruct((batch, 1), jnp.float32),
    )
    return head(u_full, m_full, urem, mrem, genre_features,
                genre_W, genre_b.reshape(1, -1), fc_W, fc_b.reshape(1, 1))
